# baseline (speedup 1.0000x reference)
import numpy as np
import jax
import jax.numpy as jnp
from jax.experimental import pallas as pl
from jax.experimental.pallas import tpu as pltpu

N = 4096
D = 128
TAU = 1.0
SIGMA = 1.2
R_EMIS = 0.5
LOG2PI = float(np.log(2.0 * np.pi))

BLK = 256
GRID = N // BLK


def _np_rotl(x, d):
    return ((x << np.uint32(d)) | (x >> np.uint32(32 - d))).astype(np.uint32)


def _np_threefry2x32(k1, k2, x0, x1):
    x0 = x0.astype(np.uint32)
    x1 = x1.astype(np.uint32)
    ks0 = np.uint32(k1)
    ks1 = np.uint32(k2)
    ks2 = np.uint32(0x1BD11BDA) ^ ks0 ^ ks1
    ks = [ks0, ks1, ks2]
    rots = [(13, 15, 26, 6), (17, 29, 16, 24)]
    x0 = (x0 + ks0).astype(np.uint32)
    x1 = (x1 + ks1).astype(np.uint32)
    for i in range(5):
        for r in rots[i % 2]:
            x0 = (x0 + x1).astype(np.uint32)
            x1 = _np_rotl(x1, r)
            x1 = x1 ^ x0
        x0 = (x0 + ks[(i + 1) % 3]).astype(np.uint32)
        x1 = (x1 + ks[(i + 2) % 3] + np.uint32(i + 1)).astype(np.uint32)
    return x0, x1


_S0, _S1 = _np_threefry2x32(0, 1, np.zeros(2, np.uint32), np.arange(2, dtype=np.uint32))
RK0, RK1 = int(_S0[0]), int(_S1[0])
PK0, PK1 = int(_S0[1]), int(_S1[1])

U_MIN = np.float32(1e-12)
U_SCALE = np.float32(1.0) - np.float32(1e-12)
N_LO = np.float32(np.nextafter(np.float32(-1.0), np.float32(0.0)))
N_SCALE = np.float32(1.0) - N_LO
SQRT2 = np.float32(np.sqrt(2.0))


def _threefry(k1, k2, x1):
    ks0 = jnp.uint32(k1)
    ks1 = jnp.uint32(k2)
    ks2 = jnp.uint32(np.uint32(0x1BD11BDA) ^ np.uint32(k1) ^ np.uint32(k2))
    ks = (ks0, ks1, ks2)
    rots = ((13, 15, 26, 6), (17, 29, 16, 24))
    x0 = jnp.full(x1.shape, ks0, jnp.uint32)
    x1 = x1 + ks1
    for i in range(5):
        for r in rots[i % 2]:
            x0 = x0 + x1
            x1 = (x1 << r) | (x1 >> (32 - r))
            x1 = x1 ^ x0
        x0 = x0 + ks[(i + 1) % 3]
        x1 = x1 + ks[(i + 2) % 3] + jnp.uint32(i + 1)
    return x0 ^ x1


def _bits_to_f01(bits):
    fb = (bits >> 9) | jnp.uint32(0x3F800000)
    return jax.lax.bitcast_convert_type(fb, jnp.float32) - jnp.float32(1.0)


def _erfinv(x):
    w = -jnp.log((jnp.float32(1.0) - x) * (jnp.float32(1.0) + x))
    ws = w - jnp.float32(2.5)
    p = jnp.float32(2.81022636e-08)
    for c in (3.43273939e-07, -3.5233877e-06, -4.39150654e-06, 0.00021858087,
              -0.00125372503, -0.00417768164, 0.246640727, 1.50140941):
        p = jnp.float32(c) + p * ws
    wl = jnp.sqrt(w) - jnp.float32(3.0)
    q = jnp.float32(-0.000200214257)
    for c in (0.000100950558, 0.00134934322, -0.00367342844, 0.00573950773,
              -0.0076224613, 0.00943887047, 1.00167406, 2.83297682):
        q = jnp.float32(c) + q * wl
    return jnp.where(w < jnp.float32(5.0), p, q) * x


def _sis_kernel(lw_ref, p_ref, obs_ref, logw_ref, next_ref, ess_ref):
    i = pl.program_id(0)
    r0 = i * BLK

    lw = lw_ref[:]
    m = jnp.max(lw)
    t = jnp.exp(lw - m)
    s1 = jnp.sum(t)
    s2 = jnp.sum(t * t)
    ess = s1 * s1 / (s2 * jnp.float32(N))
    ess_ref[...] = jnp.reshape(ess, (1, 1))
    resample = ess < jnp.float32(0.5)

    row = jax.lax.broadcasted_iota(jnp.int32, (BLK, N), 0)
    col = jax.lax.broadcasted_iota(jnp.int32, (BLK, N), 1)
    ctr = ((r0 + row) * N + col).astype(jnp.uint32)
    bits = _threefry(RK0, RK1, ctr)
    f01 = _bits_to_f01(bits)
    u = jnp.maximum(U_MIN, f01 * U_SCALE + U_MIN)
    vals = lw[None, :] + (-jnp.log(-jnp.log(u)))
    rowmax = jnp.max(vals, axis=1, keepdims=True)
    ix = jnp.min(jnp.where(vals == rowmax, col, N), axis=1)

    onehot = (col[:, :] == ix[:, None]).astype(jnp.float32)
    gathered = jax.lax.dot_general(
        onehot, p_ref[:],
        dimension_numbers=(((1,), (0,)), ((), ())),
        preferred_element_type=jnp.float32,
        precision=jax.lax.Precision.HIGHEST)
    p_blk = p_ref[pl.ds(r0, BLK), :]
    pr = jnp.where(resample, gathered, p_blk)

    ctr2 = ((r0 + jax.lax.broadcasted_iota(jnp.int32, (BLK, D), 0)) * D
            + jax.lax.broadcasted_iota(jnp.int32, (BLK, D), 1)).astype(jnp.uint32)
    f2 = _bits_to_f01(_threefry(PK0, PK1, ctr2))
    u2 = jnp.maximum(N_LO, f2 * N_SCALE + N_LO)
    eps = SQRT2 * _erfinv(u2)

    nxt = pr + jnp.float32(SIGMA) * eps
    next_ref[...] = nxt

    diff = nxt - pr
    obs = obs_ref[:]
    dobs = obs[None, :] - nxt
    half_d = jnp.float32(0.5 * D)
    trans = (-0.5 * jnp.sum((diff / jnp.float32(TAU)) ** 2, axis=1)
             - jnp.float32(D * np.log(TAU)) - half_d * jnp.float32(LOG2PI))
    emis = (-0.5 * jnp.sum((dobs / jnp.float32(R_EMIS)) ** 2, axis=1)
            - jnp.float32(D * np.log(R_EMIS)) - half_d * jnp.float32(LOG2PI))
    prop = (-0.5 * jnp.sum((diff / jnp.float32(SIGMA)) ** 2, axis=1)
            - jnp.float32(D * np.log(SIGMA)) - half_d * jnp.float32(LOG2PI))

    lw_blk = lw_ref[pl.ds(r0, BLK)]
    base = jnp.where(resample, jnp.float32(0.0), lw_blk)
    logw_ref[...] = base + trans + emis - prop


def kernel(log_weights, particles, observation):
    logw, nxt, ess = pl.pallas_call(
        _sis_kernel,
        grid=(GRID,),
        in_specs=[
            pl.BlockSpec((N,), lambda i: (0,)),
            pl.BlockSpec((N, D), lambda i: (0, 0)),
            pl.BlockSpec((D,), lambda i: (0,)),
        ],
        out_specs=[
            pl.BlockSpec((BLK,), lambda i: (i,)),
            pl.BlockSpec((BLK, D), lambda i: (i, 0)),
            pl.BlockSpec((1, 1), lambda i: (0, 0)),
        ],
        out_shape=[
            jax.ShapeDtypeStruct((N,), jnp.float32),
            jax.ShapeDtypeStruct((N, D), jnp.float32),
            jax.ShapeDtypeStruct((1, 1), jnp.float32),
        ],
    )(log_weights, particles, observation)
    return logw, nxt, ess[0, 0]
